# Initial kernel scaffold; baseline (speedup 1.0000x reference)
import jax
import jax.numpy as jnp
from jax.experimental import pallas as pl


def kernel(bounding_box):
    raise NotImplementedError("write your pallas kernel here")


# speedup vs baseline: 451.5973x; 451.5973x over previous
import jax
import jax.numpy as jnp
from jax.experimental import pallas as pl
from jax.experimental.pallas import tpu as pltpu

_V = 64
_B = 32


def _occ_kernel(bb_ref, out_ref):
    b = pl.program_id(0)
    mxx = bb_ref[b, 0]
    mxy = bb_ref[b, 1]
    mxz = bb_ref[b, 2]
    mnx = bb_ref[b, 3]
    mny = bb_ref[b, 4]
    mnz = bb_ref[b, 5]
    sx = (mxx - mnx) / 64.0
    sy = (mxy - mny) / 64.0
    sz = (mxz - mnz) / 64.0

    fi = jax.lax.broadcasted_iota(jnp.int32, (1, 1, _V, _V), 3).astype(jnp.float32)
    vy = jax.lax.broadcasted_iota(jnp.int32, (1, 1, _V, _V), 2)
    cy = (fi * sy + mny).astype(jnp.int32)
    oy = jnp.max((cy == vy).astype(jnp.float32), axis=3, keepdims=True)

    fi2 = jax.lax.broadcasted_iota(jnp.int32, (1, 1, _V, _V), 2).astype(jnp.float32)
    vz = jax.lax.broadcasted_iota(jnp.int32, (1, 1, _V, _V), 3)
    cz = (fi2 * sz + mnz).astype(jnp.int32)
    oz = jnp.max((cz == vz).astype(jnp.float32), axis=2, keepdims=True)

    fi3 = jax.lax.broadcasted_iota(jnp.int32, (1, _V, 1, _V), 3).astype(jnp.float32)
    vx = jax.lax.broadcasted_iota(jnp.int32, (1, _V, 1, _V), 1)
    cx = (fi3 * sx + mnx).astype(jnp.int32)
    ox = jnp.max((cx == vx).astype(jnp.float32), axis=3, keepdims=True)

    out_ref[...] = ox * (oy * oz)


def kernel(bounding_box):
    out = pl.pallas_call(
        _occ_kernel,
        grid=(_B,),
        in_specs=[pl.BlockSpec(memory_space=pltpu.SMEM)],
        out_specs=pl.BlockSpec((1, _V, _V, _V), lambda b: (b, 0, 0, 0)),
        out_shape=jax.ShapeDtypeStruct((_B, _V, _V, _V), jnp.float32),
        compiler_params=pltpu.CompilerParams(
            dimension_semantics=("arbitrary",),
        ),
    )(bounding_box)
    return out[..., None]
